# baseline (speedup 1.0000x reference)
import functools

import jax
import jax.numpy as jnp
from jax import lax
from jax.experimental import pallas as pl
from jax.experimental.pallas import tpu as pltpu
from jax.experimental.pallas import tpu_sc as plsc

BATCH = 16384
EMBED = 64
HIDDEN = 128
K_SC = 8192
SUB = 8

NUM_CORES = 2
NUM_SUBCORES = 16
B_PER_W = K_SC // NUM_SUBCORES
CHUNK = 64
NCHUNKS = B_PER_W // CHUNK
NSETS = 4
NQ = CHUNK // NSETS


def _sc_gather(ut_t, mt_t, idx_all):
    mesh = plsc.VectorSubcoreMesh(core_axis_name="c", subcore_axis_name="s",
                                  num_cores=NUM_CORES,
                                  num_subcores=NUM_SUBCORES)

    @functools.partial(
        pl.kernel,
        out_type=jax.ShapeDtypeStruct((NUM_CORES, K_SC, EMBED), jnp.float32),
        mesh=mesh,
        scratch_types=[
            pltpu.VMEM((B_PER_W + 16,), jnp.int32),
            pltpu.VMEM((NSETS, EMBED, 128), jnp.float32),
            pltpu.VMEM((CHUNK, EMBED), jnp.float32),
            pltpu.SemaphoreType.DMA,
            pltpu.SemaphoreType.DMA,
            pltpu.SemaphoreType.DMA,
            pltpu.SemaphoreType.DMA,
        ],
        compiler_params=pltpu.CompilerParams(needs_layout_passes=False),
    )
    def kern(ut_hbm, mt_hbm, idx_hbm, out_hbm, idx, slab, comb, s0, s1, s2,
             s3):
        c = lax.axis_index("c")
        s = lax.axis_index("s")
        base = s * B_PER_W
        sems = (s0, s1, s2, s3)
        pltpu.sync_copy(idx_hbm.at[c, pl.ds(base, B_PER_W)],
                        idx.at[pl.ds(0, B_PER_W)])
        rows = lax.iota(jnp.int32, 16)

        def fire(i, p, guard):
            def _u():
                r = idx[pl.ds(i, 16)][0]
                off = pl.multiple_of((r >> 7) * 128, 128)
                pltpu.async_copy(ut_hbm.at[:, pl.ds(off, 128)],
                                 slab.at[p], sems[p])

            def _m():
                r = idx[pl.ds(i, 16)][0]
                off = pl.multiple_of((r >> 7) * 128, 128)
                pltpu.async_copy(mt_hbm.at[:, pl.ds(off, 128)],
                                 slab.at[p], sems[p])

            g = True if guard is None else guard
            pl.when(jnp.logical_and(c == 0, g))(_u)
            pl.when(jnp.logical_and(c == 1, g))(_m)

        def wait_and_extract(i, row, p):
            pltpu.make_async_copy(ut_hbm.at[:, pl.ds(0, 128)],
                                  slab.at[p], sems[p]).wait()
            r = idx[pl.ds(i, 16)][0]
            cv = jnp.full((16,), r & 127, jnp.int32)
            ps = jnp.full((16,), p, jnp.int32)
            for g in range(EMBED // 16):
                e = plsc.load_gather(slab, [ps, rows + g * 16, cv])
                comb[row, pl.ds(g * 16, 16)] = e

        for p in range(NSETS):
            fire(p, p, None)

        def chunk_body(ch, _):
            def q_body(q, _):
                i0 = ch * CHUNK + q * NSETS
                for j in range(NSETS):
                    wait_and_extract(i0 + j, q * NSETS + j, j)
                    fire(i0 + j + NSETS, j, i0 + j + NSETS < B_PER_W)
                return 0

            lax.fori_loop(0, NQ, q_body, 0)
            dst = pl.multiple_of(base + ch * CHUNK, CHUNK)
            pltpu.sync_copy(comb, out_hbm.at[c, pl.ds(dst, CHUNK)])
            return 0

        lax.fori_loop(0, NCHUNKS, chunk_body, 0)

    return kern(ut_t, mt_t, idx_all)


def _mlp_block(u_ref, m_ref, w1_ref, b1_ref, w2_ref, b2_ref, o_ref):
    x = u_ref[0] + m_ref[0]
    h = jnp.maximum(
        jnp.dot(x, w1_ref[...], preferred_element_type=jnp.float32)
        + b1_ref[...], 0.0)
    o_ref[...] = (jnp.sum(h * w2_ref[...], axis=1, keepdims=True)
                  + b2_ref[0, 0])


def _tc_mlp(emb, W1, b1, W2, b2):
    nblk = K_SC // 1024
    blk = 1024
    return pl.pallas_call(
        _mlp_block,
        grid=(nblk,),
        in_specs=[
            pl.BlockSpec((1, blk, EMBED), lambda i: (0, i, 0)),
            pl.BlockSpec((1, blk, EMBED), lambda i: (1, i, 0)),
            pl.BlockSpec((EMBED, HIDDEN), lambda i: (0, 0)),
            pl.BlockSpec((1, HIDDEN), lambda i: (0, 0)),
            pl.BlockSpec((1, HIDDEN), lambda i: (0, 0)),
            pl.BlockSpec((1, 1), lambda i: (0, 0)),
        ],
        out_specs=pl.BlockSpec((blk, 1), lambda i: (i, 0)),
        out_shape=jax.ShapeDtypeStruct((K_SC, 1), jnp.float32),
    )(emb, emb, W1, b1.reshape(1, HIDDEN), W2.reshape(1, HIDDEN),
      b2.reshape(1, 1))


def _tc_tail_body(uref, mref, *refs):
    ublocks = refs[:SUB]
    mblocks = refs[SUB:2 * SUB]
    w1_ref, b1_ref, w2_ref, b2_ref, o_ref = refs[2 * SUB:]
    i = pl.program_id(0)
    lanes = lax.broadcasted_iota(jnp.int32, (1, 128), 1)
    xs = []
    for j in range(SUB):
        cu = uref[K_SC + i * SUB + j] & 127
        cm = mref[K_SC + i * SUB + j] & 127
        mu = (lanes == cu).astype(jnp.float32)
        mm = (lanes == cm).astype(jnp.float32)
        ru = lax.dot_general(mu, ublocks[j][...], (((1,), (1,)), ((), ())),
                             preferred_element_type=jnp.float32)
        rm = lax.dot_general(mm, mblocks[j][...], (((1,), (1,)), ((), ())),
                             preferred_element_type=jnp.float32)
        xs.append(ru + rm)
    x = jnp.concatenate(xs, axis=0)
    h = jnp.maximum(
        jnp.dot(x, w1_ref[...], preferred_element_type=jnp.float32)
        + b1_ref[...], 0.0)
    o_ref[...] = (jnp.sum(h * w2_ref[...], axis=1, keepdims=True)
                  + b2_ref[0, 0])


def _tc_tail(ut_t, mt_t, users, movies, W1, b1, W2, b2):
    n_tail = BATCH - K_SC
    grid = (n_tail // SUB,)

    def slab_spec(table_pos, j):
        def imap(i, uref, mref):
            ref = uref if table_pos == 0 else mref
            return (0, ref[K_SC + i * SUB + j] >> 7)
        return pl.BlockSpec((EMBED, 128), imap)

    in_specs = ([slab_spec(0, j) for j in range(SUB)]
                + [slab_spec(1, j) for j in range(SUB)]
                + [
                    pl.BlockSpec((EMBED, HIDDEN), lambda i, u, m: (0, 0)),
                    pl.BlockSpec((1, HIDDEN), lambda i, u, m: (0, 0)),
                    pl.BlockSpec((1, HIDDEN), lambda i, u, m: (0, 0)),
                    pl.BlockSpec((1, 1), lambda i, u, m: (0, 0)),
                ])
    grid_spec = pltpu.PrefetchScalarGridSpec(
        num_scalar_prefetch=2,
        grid=grid,
        in_specs=in_specs,
        out_specs=pl.BlockSpec((SUB, 1), lambda i, u, m: (i, 0)),
    )
    return pl.pallas_call(
        _tc_tail_body,
        grid_spec=grid_spec,
        out_shape=jax.ShapeDtypeStruct((n_tail, 1), jnp.float32),
    )(users, movies, *([ut_t] * SUB), *([mt_t] * SUB),
      W1, b1.reshape(1, HIDDEN), W2.reshape(1, HIDDEN), b2.reshape(1, 1))


@jax.jit
def kernel(users, movies, user_table, movie_table, W1, b1, W2, b2):
    ut_t = jnp.transpose(user_table)
    mt_t = jnp.transpose(movie_table)
    users = users.astype(jnp.int32)
    movies = movies.astype(jnp.int32)
    idx_all = jnp.stack([users, movies])
    emb = _sc_gather(ut_t, mt_t, idx_all)
    head = _tc_mlp(emb, W1, b1, W2, b2)
    tail = _tc_tail(ut_t, mt_t, users, movies, W1, b1, W2, b2)
    return jnp.concatenate([head, tail], axis=0)


# speedup vs baseline: 1.3483x; 1.3483x over previous
import functools

import jax
import jax.numpy as jnp
from jax import lax
from jax.experimental import pallas as pl
from jax.experimental.pallas import tpu as pltpu
from jax.experimental.pallas import tpu_sc as plsc

BATCH = 16384
EMBED = 64
HIDDEN = 128
K_SC = 8192
SUB = 16

NUM_CORES = 2
NUM_SUBCORES = 16
B_PER_W = K_SC // NUM_SUBCORES
CHUNK = 64
NCHUNKS = B_PER_W // CHUNK
NSETS = 4
NQ = CHUNK // NSETS


def _sc_gather(ut_t, mt_t, idx_all):
    mesh = plsc.VectorSubcoreMesh(core_axis_name="c", subcore_axis_name="s",
                                  num_cores=NUM_CORES,
                                  num_subcores=NUM_SUBCORES)

    @functools.partial(
        pl.kernel,
        out_type=jax.ShapeDtypeStruct((NUM_CORES, K_SC, EMBED), jnp.float32),
        mesh=mesh,
        scratch_types=[
            pltpu.VMEM((B_PER_W + 16,), jnp.int32),
            pltpu.VMEM((NSETS, EMBED, 128), jnp.float32),
            pltpu.VMEM((CHUNK, EMBED), jnp.float32),
            pltpu.SemaphoreType.DMA,
            pltpu.SemaphoreType.DMA,
            pltpu.SemaphoreType.DMA,
            pltpu.SemaphoreType.DMA,
        ],
        compiler_params=pltpu.CompilerParams(needs_layout_passes=False),
    )
    def kern(ut_hbm, mt_hbm, idx_hbm, out_hbm, idx, slab, comb, s0, s1, s2,
             s3):
        c = lax.axis_index("c")
        s = lax.axis_index("s")
        base = s * B_PER_W
        sems = (s0, s1, s2, s3)
        pltpu.sync_copy(idx_hbm.at[c, pl.ds(base, B_PER_W)],
                        idx.at[pl.ds(0, B_PER_W)])
        rows = lax.iota(jnp.int32, 16)

        def fire(i, p, guard):
            def _u():
                r = idx[pl.ds(i, 16)][0]
                off = pl.multiple_of((r >> 7) * 128, 128)
                pltpu.async_copy(ut_hbm.at[:, pl.ds(off, 128)],
                                 slab.at[p], sems[p])

            def _m():
                r = idx[pl.ds(i, 16)][0]
                off = pl.multiple_of((r >> 7) * 128, 128)
                pltpu.async_copy(mt_hbm.at[:, pl.ds(off, 128)],
                                 slab.at[p], sems[p])

            g = True if guard is None else guard
            pl.when(jnp.logical_and(c == 0, g))(_u)
            pl.when(jnp.logical_and(c == 1, g))(_m)

        def wait_and_extract(i, row, p):
            pltpu.make_async_copy(ut_hbm.at[:, pl.ds(0, 128)],
                                  slab.at[p], sems[p]).wait()
            r = idx[pl.ds(i, 16)][0]
            cv = jnp.full((16,), r & 127, jnp.int32)
            ps = jnp.full((16,), p, jnp.int32)
            for g in range(EMBED // 16):
                e = plsc.load_gather(slab, [ps, rows + g * 16, cv])
                comb[row, pl.ds(g * 16, 16)] = e

        for p in range(NSETS):
            fire(p, p, None)

        def chunk_body(ch, _):
            def q_body(q, _):
                i0 = ch * CHUNK + q * NSETS
                for j in range(NSETS):
                    wait_and_extract(i0 + j, q * NSETS + j, j)
                    fire(i0 + j + NSETS, j, i0 + j + NSETS < B_PER_W)
                return 0

            lax.fori_loop(0, NQ, q_body, 0)
            dst = pl.multiple_of(base + ch * CHUNK, CHUNK)
            pltpu.sync_copy(comb, out_hbm.at[c, pl.ds(dst, CHUNK)])
            return 0

        lax.fori_loop(0, NCHUNKS, chunk_body, 0)

    return kern(ut_t, mt_t, idx_all)


def _mlp_block(u_ref, m_ref, w1_ref, b1_ref, w2_ref, b2_ref, o_ref):
    x = u_ref[0] + m_ref[0]
    h = jnp.maximum(
        jnp.dot(x, w1_ref[...], preferred_element_type=jnp.float32)
        + b1_ref[...], 0.0)
    o_ref[...] = (jnp.sum(h * w2_ref[...], axis=1, keepdims=True)
                  + b2_ref[0, 0])


def _tc_mlp(emb, W1, b1, W2, b2):
    nblk = K_SC // 1024
    blk = 1024
    return pl.pallas_call(
        _mlp_block,
        grid=(nblk,),
        in_specs=[
            pl.BlockSpec((1, blk, EMBED), lambda i: (0, i, 0)),
            pl.BlockSpec((1, blk, EMBED), lambda i: (1, i, 0)),
            pl.BlockSpec((EMBED, HIDDEN), lambda i: (0, 0)),
            pl.BlockSpec((1, HIDDEN), lambda i: (0, 0)),
            pl.BlockSpec((1, HIDDEN), lambda i: (0, 0)),
            pl.BlockSpec((1, 1), lambda i: (0, 0)),
        ],
        out_specs=pl.BlockSpec((blk, 1), lambda i: (i, 0)),
        out_shape=jax.ShapeDtypeStruct((K_SC, 1), jnp.float32),
    )(emb, emb, W1, b1.reshape(1, HIDDEN), W2.reshape(1, HIDDEN),
      b2.reshape(1, 1))


def _tc_tail_body(uref, mref, *refs):
    ublocks = refs[:SUB]
    mblocks = refs[SUB:2 * SUB]
    w1_ref, b1_ref, w2_ref, b2_ref, o_ref = refs[2 * SUB:]
    i = pl.program_id(0)
    lanes = lax.broadcasted_iota(jnp.int32, (1, 128), 1)
    xs = []
    for j in range(SUB):
        cu = uref[K_SC + i * SUB + j] & 127
        cm = mref[K_SC + i * SUB + j] & 127
        mu = (lanes == cu).astype(jnp.float32)
        mm = (lanes == cm).astype(jnp.float32)
        ru = lax.dot_general(mu, ublocks[j][...], (((1,), (1,)), ((), ())),
                             preferred_element_type=jnp.float32)
        rm = lax.dot_general(mm, mblocks[j][...], (((1,), (1,)), ((), ())),
                             preferred_element_type=jnp.float32)
        xs.append(ru + rm)
    x = jnp.concatenate(xs, axis=0)
    h = jnp.maximum(
        jnp.dot(x, w1_ref[...], preferred_element_type=jnp.float32)
        + b1_ref[...], 0.0)
    o_ref[...] = (jnp.sum(h * w2_ref[...], axis=1, keepdims=True)
                  + b2_ref[0, 0])


def _tc_tail(ut_t, mt_t, users, movies, W1, b1, W2, b2):
    n_tail = BATCH - K_SC
    grid = (n_tail // SUB,)

    def slab_spec(table_pos, j):
        def imap(i, uref, mref):
            ref = uref if table_pos == 0 else mref
            return (0, ref[K_SC + i * SUB + j] >> 7)
        return pl.BlockSpec((EMBED, 128), imap)

    in_specs = ([slab_spec(0, j) for j in range(SUB)]
                + [slab_spec(1, j) for j in range(SUB)]
                + [
                    pl.BlockSpec((EMBED, HIDDEN), lambda i, u, m: (0, 0)),
                    pl.BlockSpec((1, HIDDEN), lambda i, u, m: (0, 0)),
                    pl.BlockSpec((1, HIDDEN), lambda i, u, m: (0, 0)),
                    pl.BlockSpec((1, 1), lambda i, u, m: (0, 0)),
                ])
    grid_spec = pltpu.PrefetchScalarGridSpec(
        num_scalar_prefetch=2,
        grid=grid,
        in_specs=in_specs,
        out_specs=pl.BlockSpec((SUB, 1), lambda i, u, m: (i, 0)),
    )
    return pl.pallas_call(
        _tc_tail_body,
        grid_spec=grid_spec,
        out_shape=jax.ShapeDtypeStruct((n_tail, 1), jnp.float32),
    )(users, movies, *([ut_t] * SUB), *([mt_t] * SUB),
      W1, b1.reshape(1, HIDDEN), W2.reshape(1, HIDDEN), b2.reshape(1, 1))


@jax.jit
def kernel(users, movies, user_table, movie_table, W1, b1, W2, b2):
    ut_t = jnp.transpose(user_table)
    mt_t = jnp.transpose(movie_table)
    users = users.astype(jnp.int32)
    movies = movies.astype(jnp.int32)
    idx_all = jnp.stack([users, movies])
    emb = _sc_gather(ut_t, mt_t, idx_all)
    head = _tc_mlp(emb, W1, b1, W2, b2)
    tail = _tc_tail(ut_t, mt_t, users, movies, W1, b1, W2, b2)
    return jnp.concatenate([head, tail], axis=0)


# speedup vs baseline: 1.8140x; 1.3454x over previous
import functools

import jax
import jax.numpy as jnp
from jax import lax
from jax.experimental import pallas as pl
from jax.experimental.pallas import tpu as pltpu
from jax.experimental.pallas import tpu_sc as plsc

BATCH = 16384
EMBED = 64
HIDDEN = 128

NUM_CORES = 2
NUM_SUBCORES = 16
B_PER_W = BATCH // NUM_SUBCORES
CHUNK = 64
NCHUNKS = B_PER_W // CHUNK
NSETS = 4
NQ = CHUNK // NSETS


def _sc_gather(ut_t, mt_t, idx_all):
    mesh = plsc.VectorSubcoreMesh(core_axis_name="c", subcore_axis_name="s",
                                  num_cores=NUM_CORES,
                                  num_subcores=NUM_SUBCORES)

    @functools.partial(
        pl.kernel,
        out_type=jax.ShapeDtypeStruct((NUM_CORES, BATCH, EMBED),
                                      jnp.float32),
        mesh=mesh,
        scratch_types=[
            pltpu.VMEM((B_PER_W + 16,), jnp.int32),
            pltpu.VMEM((NSETS, EMBED, 128), jnp.float32),
            pltpu.VMEM((CHUNK, EMBED), jnp.float32),
            pltpu.SemaphoreType.DMA,
            pltpu.SemaphoreType.DMA,
            pltpu.SemaphoreType.DMA,
            pltpu.SemaphoreType.DMA,
        ],
        compiler_params=pltpu.CompilerParams(needs_layout_passes=False),
    )
    def kern(ut_hbm, mt_hbm, idx_hbm, out_hbm, idx, slab, comb, s0, s1, s2,
             s3):
        c = lax.axis_index("c")
        s = lax.axis_index("s")
        base = s * B_PER_W
        sems = (s0, s1, s2, s3)
        pltpu.sync_copy(idx_hbm.at[c, pl.ds(base, B_PER_W)],
                        idx.at[pl.ds(0, B_PER_W)])
        rows = lax.iota(jnp.int32, 16)

        def fire(r, p, guard):
            def _u():
                off = pl.multiple_of((r >> 7) * 128, 128)
                pltpu.async_copy(ut_hbm.at[:, pl.ds(off, 128)],
                                 slab.at[p], sems[p])

            def _m():
                off = pl.multiple_of((r >> 7) * 128, 128)
                pltpu.async_copy(mt_hbm.at[:, pl.ds(off, 128)],
                                 slab.at[p], sems[p])

            g = True if guard is None else guard
            pl.when(jnp.logical_and(c == 0, g))(_u)
            pl.when(jnp.logical_and(c == 1, g))(_m)

        def wait_and_extract(r, row, p):
            pltpu.make_async_copy(ut_hbm.at[:, pl.ds(0, 128)],
                                  slab.at[p], sems[p]).wait()
            cv = jnp.full((16,), r & 127, jnp.int32)
            ps = jnp.full((16,), p, jnp.int32)
            for g in range(EMBED // 16):
                e = plsc.load_gather(slab, [ps, rows + g * 16, cv])
                comb[row, pl.ds(g * 16, 16)] = e

        vec0 = idx[pl.ds(0, 16)]
        for p in range(NSETS):
            fire(vec0[p], p, None)

        def chunk_body(ch, _):
            def q_body(q, _):
                i0 = ch * CHUNK + q * NSETS
                vec = idx[pl.ds(i0, 16)]
                for j in range(NSETS):
                    wait_and_extract(vec[j], q * NSETS + j, j)
                    fire(vec[NSETS + j], j, i0 + j + NSETS < B_PER_W)
                return 0

            lax.fori_loop(0, NQ, q_body, 0)
            dst = pl.multiple_of(base + ch * CHUNK, CHUNK)
            pltpu.sync_copy(comb, out_hbm.at[c, pl.ds(dst, CHUNK)])
            return 0

        lax.fori_loop(0, NCHUNKS, chunk_body, 0)

    return kern(ut_t, mt_t, idx_all)


def _mlp_block(u_ref, m_ref, w1_ref, b1_ref, w2_ref, b2_ref, o_ref):
    x = u_ref[0] + m_ref[0]
    h = jnp.maximum(
        jnp.dot(x, w1_ref[...], preferred_element_type=jnp.float32)
        + b1_ref[...], 0.0)
    o_ref[...] = (jnp.sum(h * w2_ref[...], axis=1, keepdims=True)
                  + b2_ref[0, 0])


def _tc_mlp(emb, W1, b1, W2, b2):
    nblk = 16
    blk = BATCH // nblk
    return pl.pallas_call(
        _mlp_block,
        grid=(nblk,),
        in_specs=[
            pl.BlockSpec((1, blk, EMBED), lambda i: (0, i, 0)),
            pl.BlockSpec((1, blk, EMBED), lambda i: (1, i, 0)),
            pl.BlockSpec((EMBED, HIDDEN), lambda i: (0, 0)),
            pl.BlockSpec((1, HIDDEN), lambda i: (0, 0)),
            pl.BlockSpec((1, HIDDEN), lambda i: (0, 0)),
            pl.BlockSpec((1, 1), lambda i: (0, 0)),
        ],
        out_specs=pl.BlockSpec((blk, 1), lambda i: (i, 0)),
        out_shape=jax.ShapeDtypeStruct((BATCH, 1), jnp.float32),
    )(emb, emb, W1, b1.reshape(1, HIDDEN), W2.reshape(1, HIDDEN),
      b2.reshape(1, 1))


@jax.jit
def kernel(users, movies, user_table, movie_table, W1, b1, W2, b2):
    ut_t = jnp.transpose(user_table)
    mt_t = jnp.transpose(movie_table)
    idx_all = jnp.stack([users.astype(jnp.int32), movies.astype(jnp.int32)])
    emb = _sc_gather(ut_t, mt_t, idx_all)
    return _tc_mlp(emb, W1, b1, W2, b2)
